# Initial kernel scaffold; baseline (speedup 1.0000x reference)
import jax
import jax.numpy as jnp
from jax.experimental import pallas as pl


def kernel(left_score, right_score, batch_decision_score, batch_unary_score, sent_lens):
    raise NotImplementedError("write your pallas kernel here")


# speedup vs baseline: 8.3703x; 8.3703x over previous
import jax
import jax.numpy as jnp
from jax.experimental import pallas as pl
from jax.experimental.pallas import tpu as pltpu

_NEG = -1e9
_B, _S, _T, _V = 16, 24, 32, 2


_BH = 8


def _dp_kernel(rs0, rs1, ls0, ls1, bdt, u, af,
               c0d0, c0d1, c1d0, c1d1, hd, g0, g1):
    s, b, t = _S, _BH, _T
    f32 = jnp.float32

    rsm = jnp.maximum(rs0[...], rs1[...])
    lsm = jnp.maximum(ls0[...], ls1[...])
    mr = jnp.max(rsm, axis=(2, 3))
    ml = jnp.max(lsm, axis=(2, 3))

    ub = u[...]
    c0d0[0] = bdt[:, 0, 0, 0] + ub
    c0d1[0] = bdt[:, 0, 1, 0] + ub
    c1d0[0] = bdt[:, 1, 0, 0] + ub
    c1d1[0] = bdt[:, 1, 1, 0] + ub
    hd[0] = c0d0[0] + c1d0[0]

    for ln in range(1, s):
        ni = s - ln

        def mx_body(q, carry):
            mq0, mq1 = carry
            x0 = jnp.max(hd[q, 0:ni], axis=2)
            y0 = jnp.max(c0d1[ln - 1 - q, pl.ds(1 + q, ni)], axis=2)
            x1 = jnp.max(c1d1[q, 0:ni], axis=2)
            y1 = jnp.max(hd[ln - 1 - q, pl.ds(1 + q, ni)], axis=2)
            return (jnp.maximum(mq0, x0 + y0), jnp.maximum(mq1, x1 + y1))

        init = tuple(jnp.full((ni, b), _NEG, f32) for _ in range(2))
        mq0, mq1 = jax.lax.fori_loop(0, ln, mx_body, init)

        g0[0:ni] = jnp.zeros((ni, b, t, t), f32)
        g1[0:ni] = jnp.zeros((ni, b, t, t), f32)

        def acc_body(q, _):
            x0 = hd[q, 0:ni]
            y0 = c0d1[ln - 1 - q, pl.ds(1 + q, ni)]
            a0 = jnp.max(x0, axis=2)[:, :, None]
            ex0 = jnp.exp(x0 - a0)
            ey0 = jnp.exp(y0 + a0 - mq0[:, :, None])
            g0[0:ni] += ex0[:, :, :, None] * ey0[:, :, None, :]
            x1 = c1d1[q, 0:ni]
            y1 = hd[ln - 1 - q, pl.ds(1 + q, ni)]
            a1 = jnp.max(x1, axis=2)[:, :, None]
            ex1 = jnp.exp(x1 - a1)
            ey1 = jnp.exp(y1 + a1 - mq1[:, :, None])
            g1[0:ni] += ex1[:, :, :, None] * ey1[:, :, None, :]
            return 0

        jax.lax.fori_loop(0, ln, acc_body, 0)

        G0 = g0[0:ni]
        G1 = g1[0:ni]
        sh0 = (mq0 + mr[ln:s])[:, :, None]
        mrb = mr[ln:s][:, :, None, None]
        s00 = jnp.sum(jnp.exp(rs0[ln:s] - mrb) * G0, axis=2)
        s01 = jnp.sum(jnp.exp(rs1[ln:s] - mrb) * G0, axis=2)
        c0d0[ln, 0:ni] = bdt[ln:s, 0, 0, 1] + sh0 + jnp.log(s00)
        c0d1[ln, 0:ni] = bdt[ln:s, 0, 1, 1] + sh0 + jnp.log(s01)
        sh1 = (mq1 + ml[0:ni])[:, :, None]
        mlb = ml[0:ni][:, :, None, None]
        s10 = jnp.sum(jnp.exp(ls0[0:ni] - mlb) * G1, axis=3)
        s11 = jnp.sum(jnp.exp(ls1[0:ni] - mlb) * G1, axis=3)
        c1d0[ln, 0:ni] = bdt[0:ni, 1, 0, 1] + sh1 + jnp.log(s10)
        c1d1[ln, 0:ni] = bdt[0:ni, 1, 1, 1] + sh1 + jnp.log(s11)

        def hm_body(k, mz):
            z = c0d0[k, 0:ni] + c1d0[ln - k, pl.ds(k, ni)]
            return jnp.maximum(mz, z)

        mz = jax.lax.fori_loop(0, ln + 1, hm_body, jnp.full((ni, b, t), _NEG, f32))

        def hacc_body(k, acc):
            z = c0d0[k, 0:ni] + c1d0[ln - k, pl.ds(k, ni)]
            return acc + jnp.exp(z - mz)

        sha = jax.lax.fori_loop(0, ln + 1, hacc_body, jnp.zeros((ni, b, t), f32))
        hd[ln, 0:ni] = mz + jnp.log(sha)

    af[0] = c1d0[:, 0, :, 0]


def _run_dp(rs0, rs1, ls0, ls1, bdt, u):
    s, b, t = _S, _BH, _T
    f32 = jnp.float32
    scratch = [
        pltpu.VMEM((s, s, b, t), f32),
        pltpu.VMEM((s, s, b, t), f32),
        pltpu.VMEM((s, s, b, t), f32),
        pltpu.VMEM((s, s, b, t), f32),
        pltpu.VMEM((s, s, b, t), f32),
        pltpu.VMEM((s - 1, b, t, t), f32),
        pltpu.VMEM((s - 1, b, t, t), f32),
    ]
    score_spec = pl.BlockSpec((s, b, t, t), lambda h: (0, h, 0, 0))
    return pl.pallas_call(
        _dp_kernel,
        grid=(_B // _BH,),
        in_specs=[
            score_spec, score_spec, score_spec, score_spec,
            pl.BlockSpec((s, 2, 2, 2, b, t), lambda h: (0, 0, 0, 0, h, 0)),
            pl.BlockSpec((s, b, t), lambda h: (0, h, 0)),
        ],
        out_specs=pl.BlockSpec((1, s, b), lambda h: (h, 0, 0)),
        out_shape=jax.ShapeDtypeStruct((_B // _BH, s, b), f32),
        scratch_shapes=scratch,
    )(rs0, rs1, ls0, ls1, bdt, u)


def kernel(left_score, right_score, batch_decision_score, batch_unary_score, sent_lens):
    b, s, t, _ = left_score.shape
    ls5 = left_score.reshape(b, s, t, t, _V)
    rs5 = right_score.reshape(b, s, t, t, _V)
    rs0 = rs5[..., 0].transpose(1, 0, 2, 3)
    rs1 = rs5[..., 1].transpose(1, 0, 2, 3)
    ls0 = ls5[..., 0].transpose(1, 0, 2, 3)
    ls1 = ls5[..., 1].transpose(1, 0, 2, 3)
    bdt = batch_decision_score.transpose(1, 3, 4, 5, 0, 2)
    u = batch_unary_score.transpose(1, 0, 2)
    af3 = _run_dp(rs0, rs1, ls0, ls1, bdt, u)
    af = jnp.moveaxis(af3, 0, 1).reshape(s, b)
    return af[sent_lens - 1, jnp.arange(b)]


# speedup vs baseline: 15.4625x; 1.8473x over previous
import jax
import jax.numpy as jnp
from jax.experimental import pallas as pl
from jax.experimental.pallas import tpu as pltpu

_NEG = -1e9
_B, _S, _T, _V = 16, 24, 32, 2
_L = _S * _B


def _dp_kernel(rst0, rst1, lst0, lst1, bd, u, af,
               c0d0t, c0d1e, c1d0e, c1d1t, hdt, hde, er0, er1, el0, el1, g0, g1):
    s, t, L = _S, _T, _L
    f32 = jnp.float32
    roll = lambda x, k: pltpu.roll(x, k % L, axis=x.ndim - 1)

    mrx = jnp.max(jnp.maximum(rst0[...], rst1[...]), axis=0)
    er0[...] = jnp.exp(rst0[...] - mrx[None])
    er1[...] = jnp.exp(rst1[...] - mrx[None])
    mlx = jnp.max(jnp.maximum(lst0[...], lst1[...]), axis=0)
    el0[...] = jnp.exp(lst0[...] - mlx[None])
    el1[...] = jnp.exp(lst1[...] - mlx[None])

    ub = u[...]
    c0d0t[0] = bd[0, 0, 0] + ub
    c0d1e[0] = bd[0, 1, 0] + ub
    c1d0e[0] = bd[1, 0, 0] + ub
    c1d1t[0] = bd[1, 1, 0] + ub
    h0 = c0d0t[0] + c1d0e[0]
    hdt[0] = h0
    hde[0] = h0
    af[0] = (bd[1, 0, 0] + ub)[0, 0:_B]

    for ln in range(1, s):
        r = ln * _B

        def mx_body(q, carry):
            mq0, mq1 = carry
            a0 = jnp.max(hdt[q], axis=0, keepdims=True)
            b0 = jnp.max(roll(c0d1e[ln - 1 - q], -r), axis=0, keepdims=True)
            a1 = jnp.max(c1d1t[q], axis=0, keepdims=True)
            b1 = jnp.max(roll(hde[ln - 1 - q], -r), axis=0, keepdims=True)
            return jnp.maximum(mq0, a0 + b0), jnp.maximum(mq1, a1 + b1)

        init = (jnp.full((1, L), _NEG, f32), jnp.full((1, L), _NEG, f32))
        mq0, mq1 = jax.lax.fori_loop(0, ln, mx_body, init)

        g0[...] = jnp.zeros((t, t, L), f32)
        g1[...] = jnp.zeros((t, t, L), f32)

        def acc_body(q, _):
            x0 = hdt[q]
            a0 = jnp.max(x0, axis=0, keepdims=True)
            ex0 = jnp.exp(x0 - a0)
            y0 = roll(c0d1e[ln - 1 - q], -r)
            ey0 = jnp.exp(y0 + a0 - mq0)
            g0[...] += jnp.broadcast_to(ex0[:, None], (t, t, L)) * ey0[None]
            x1 = c1d1t[q]
            a1 = jnp.max(x1, axis=0, keepdims=True)
            ex1 = jnp.exp(x1 - a1)
            y1 = roll(hde[ln - 1 - q], -r)
            ey1 = jnp.exp(y1 + a1 - mq1)
            g1[...] += jnp.broadcast_to(ey1[:, None], (t, t, L)) * ex1[None]
            return 0

        jax.lax.fori_loop(0, ln, acc_body, 0)

        g0e = roll(g0[...], r)
        sh0 = roll(mq0, r) + mrx
        s00 = jnp.sum(er0[...] * g0e, axis=0)
        s01 = jnp.sum(er1[...] * g0e, axis=0)
        out00 = bd[0, 0, 1] + sh0 + jnp.log(s00)
        out01 = bd[0, 1, 1] + sh0 + jnp.log(s01)
        c0d0t[ln] = roll(out00, -r)
        c0d1e[ln] = out01
        G1 = g1[...]
        sh1 = mq1 + mlx
        s10 = jnp.sum(el0[...] * G1, axis=0)
        s11 = jnp.sum(el1[...] * G1, axis=0)
        out10 = bd[1, 0, 1] + sh1 + jnp.log(s10)
        out11 = bd[1, 1, 1] + sh1 + jnp.log(s11)
        c1d0e[ln] = roll(out10, r)
        c1d1t[ln] = out11
        af[ln] = out10[0, 0:_B]

        def hm_body(k, mz):
            return jnp.maximum(mz, c0d0t[k] + roll(c1d0e[ln - k], -r))

        mz = jax.lax.fori_loop(0, ln + 1, hm_body, jnp.full((t, L), _NEG, f32))

        def hacc_body(k, acc):
            return acc + jnp.exp(c0d0t[k] + roll(c1d0e[ln - k], -r) - mz)

        sha = jax.lax.fori_loop(0, ln + 1, hacc_body, jnp.zeros((t, L), f32))
        hrow = mz + jnp.log(sha)
        hdt[ln] = hrow
        hde[ln] = roll(hrow, r)


def _run_dp(rst0, rst1, lst0, lst1, bd, u):
    s, t, L = _S, _T, _L
    f32 = jnp.float32
    scratch = [
        pltpu.VMEM((s, t, L), f32),
        pltpu.VMEM((s, t, L), f32),
        pltpu.VMEM((s, t, L), f32),
        pltpu.VMEM((s, t, L), f32),
        pltpu.VMEM((s, t, L), f32),
        pltpu.VMEM((s, t, L), f32),
        pltpu.VMEM((t, t, L), f32),
        pltpu.VMEM((t, t, L), f32),
        pltpu.VMEM((t, t, L), f32),
        pltpu.VMEM((t, t, L), f32),
        pltpu.VMEM((t, t, L), f32),
        pltpu.VMEM((t, t, L), f32),
    ]
    return pl.pallas_call(
        _dp_kernel,
        out_shape=jax.ShapeDtypeStruct((s, _B), f32),
        scratch_shapes=scratch,
    )(rst0, rst1, lst0, lst1, bd, u)


def kernel(left_score, right_score, batch_decision_score, batch_unary_score, sent_lens):
    b, s, t, _ = left_score.shape
    ls5 = left_score.reshape(b, s, t, t, _V)
    rs5 = right_score.reshape(b, s, t, t, _V)
    rst0 = rs5[..., 0].transpose(2, 3, 1, 0).reshape(t, t, s * b)
    rst1 = rs5[..., 1].transpose(2, 3, 1, 0).reshape(t, t, s * b)
    lst0 = ls5[..., 0].transpose(3, 2, 1, 0).reshape(t, t, s * b)
    lst1 = ls5[..., 1].transpose(3, 2, 1, 0).reshape(t, t, s * b)
    bd = batch_decision_score.transpose(3, 4, 5, 2, 1, 0).reshape(2, _V, 2, t, s * b)
    u = batch_unary_score.transpose(2, 1, 0).reshape(t, s * b)
    af = _run_dp(rst0, rst1, lst0, lst1, bd, u)
    return af[sent_lens - 1, jnp.arange(b)]
